# baseline (speedup 1.0000x reference)
import functools

import jax
import jax.numpy as jnp
from jax import lax
from jax.experimental import pallas as pl
from jax.experimental.pallas import tpu as pltpu
from jax.experimental.pallas import tpu_sc as plsc

N_SC_CORES = 2
N_SUBCORES = 16
NW = N_SC_CORES * N_SUBCORES
KG = 5
KC = 10
KS = 5
NCHUNK = 5

D_FEAT = 128
D_CPACK = 32
EDGES_PER_CROW = 128 // D_CPACK
CH = 80
BLOCK_E = 6400


def _sc_gather(f_y, big_table, idx, ci):
    E = idx.shape[0]
    per_w = E // NW
    n_g = per_w // CH
    ci_per_w = 2 * per_w
    n_c = ci_per_w // CH
    slots_per_block = 2 * BLOCK_E
    rows_per_block = BLOCK_E // EDGES_PER_CROW
    mesh = plsc.VectorSubcoreMesh(core_axis_name="c", subcore_axis_name="s")

    @functools.partial(
        pl.kernel,
        out_type=[
            jax.ShapeDtypeStruct((E, D_FEAT), jnp.float32),
            jax.ShapeDtypeStruct((E // EDGES_PER_CROW, 128), jnp.float32),
        ],
        mesh=mesh,
        scratch_types=[
            pltpu.VMEM((per_w,), jnp.int32),
            pltpu.VMEM((ci_per_w,), jnp.int32),
            pltpu.VMEM((KG, CH, D_FEAT), jnp.float32),
            pltpu.VMEM((KC, CH, 16), jnp.float32),
            pltpu.SemaphoreType.DMA,
            pltpu.SemaphoreType.DMA,
        ],
        compiler_params=pltpu.CompilerParams(use_tc_tiling_on_sc=False),
    )
    def gather_kernel(fy_hbm, bt_hbm, idx_hbm, ci_hbm, g_hbm, c_hbm,
                      idx_v, ci_v, gr, cr, sem_ld, sem_st):
        wid = lax.axis_index("s") * N_SC_CORES + lax.axis_index("c")
        base = wid * per_w
        cbase = wid * ci_per_w
        pltpu.sync_copy(idx_hbm.at[pl.ds(base, per_w)], idx_v)
        pltpu.sync_copy(ci_hbm.at[pl.ds(cbase, ci_per_w)], ci_v)

        @pl.loop(0, n_g // KG)
        def _(t):
            off0 = t * (KG * CH)
            for b in range(KG):
                pltpu.async_copy(
                    fy_hbm.at[idx_v.at[pl.ds(off0 + b * CH, CH)]],
                    gr.at[b], sem_ld)
            for b in range(KG):
                pltpu.make_async_copy(
                    fy_hbm.at[idx_v.at[pl.ds(0, CH)]], gr.at[b],
                    sem_ld).wait()
            for b in range(KG):
                pltpu.async_copy(
                    gr.at[b], g_hbm.at[pl.ds(base + off0 + b * CH, CH)],
                    sem_st)
            for b in range(KG):
                pltpu.make_async_copy(
                    gr.at[b], g_hbm.at[pl.ds(base, CH)], sem_st).wait()

        @pl.loop(0, n_c // KC)
        def _(t):
            off0 = t * (KC * CH)
            for b in range(KC):
                pltpu.async_copy(
                    bt_hbm.at[ci_v.at[pl.ds(off0 + b * CH, CH)]],
                    cr.at[b], sem_ld)
            for b in range(KC):
                pltpu.make_async_copy(
                    bt_hbm.at[ci_v.at[pl.ds(0, CH)]], cr.at[b],
                    sem_ld).wait()
            for b in range(KC):
                g0 = cbase + off0 + b * CH
                blk = g0 // slots_per_block
                rem = g0 % slots_per_block
                u = rem // rows_per_block
                r0 = rem % rows_per_block
                pltpu.async_copy(
                    cr.at[b],
                    c_hbm.at[pl.ds(blk * rows_per_block + r0, CH),
                             pl.ds(u * 16, 16)], sem_st)
            for b in range(KC):
                pltpu.make_async_copy(
                    cr.at[b], c_hbm.at[pl.ds(0, CH), pl.ds(0, 16)],
                    sem_st).wait()

    return gather_kernel(f_y, big_table, idx, ci)


def _gelu_bf16(x):
    xb = x.astype(jnp.bfloat16)
    c = jnp.bfloat16(0.7978845608028654)
    a = jnp.bfloat16(0.044715)
    t = jnp.tanh(c * (xb + a * (xb * xb * xb)))
    return jnp.bfloat16(0.5) * xb * (jnp.bfloat16(1.0) + t)


def _tc_mlp(G, C, W1all, W2, W3, b1, b2, b3):
    E = G.shape[0]
    H = W2.shape[0]
    bc = BLOCK_E // EDGES_PER_CROW

    def body(g_ref, c_ref, w1_ref, w2_ref, w3_ref,
             b1_ref, b2_ref, b3_ref, o_ref):
        g = g_ref[...]
        gb = g.astype(jnp.bfloat16)
        cpb = c_ref[...].astype(jnp.bfloat16)
        rep = jnp.concatenate(
            [cpb[:, 16 * u:16 * (u + 1)] for u in range(4)], axis=0)
        slf = jnp.concatenate(
            [cpb[:, 16 * u:16 * (u + 1)] for u in range(4, 8)], axis=0)
        x1 = jnp.concatenate(
            [rep, slf, gb,
             jnp.zeros((BLOCK_E, 96), jnp.bfloat16)], axis=1)
        h = jnp.dot(x1, w1_ref[...],
                    preferred_element_type=jnp.float32) + b1_ref[...]
        h = _gelu_bf16(h)
        h = jnp.dot(h, w2_ref[...],
                    preferred_element_type=jnp.float32) + b2_ref[...]
        h = _gelu_bf16(h)
        k = jnp.dot(h, w3_ref[...], preferred_element_type=jnp.float32)
        o_ref[...] = (k + b3_ref[...]) * g

    return pl.pallas_call(
        body,
        grid=(E // BLOCK_E,),
        in_specs=[
            pl.BlockSpec((BLOCK_E, D_FEAT), lambda i: (i, 0)),
            pl.BlockSpec((bc, 128), lambda i: (i, 0)),
            pl.BlockSpec((256, H), lambda i: (0, 0)),
            pl.BlockSpec((H, H), lambda i: (0, 0)),
            pl.BlockSpec((H, D_FEAT), lambda i: (0, 0)),
            pl.BlockSpec((1, H), lambda i: (0, 0)),
            pl.BlockSpec((1, H), lambda i: (0, 0)),
            pl.BlockSpec((1, D_FEAT), lambda i: (0, 0)),
        ],
        out_specs=pl.BlockSpec((BLOCK_E, D_FEAT), lambda i: (i, 0)),
        out_shape=jax.ShapeDtypeStruct((E, D_FEAT), jnp.float32),
        compiler_params=pltpu.CompilerParams(
            dimension_semantics=("parallel",)),
    )(G, C, W1all, W2, W3, b1, b2, b3)


def _sc_scatter(k_arr, seg_ord, m):
    E = k_arr.shape[0]
    per_core = E // N_SC_CORES
    per_w = per_core // N_SUBCORES
    chs = 40
    n_ch = per_w // chs
    rows_per_tile = m // N_SUBCORES
    mesh = plsc.VectorSubcoreMesh(core_axis_name="c", subcore_axis_name="s")
    zeros = jnp.zeros((rows_per_tile, D_FEAT), jnp.float32)

    @functools.partial(
        pl.kernel,
        out_type=jax.ShapeDtypeStruct((N_SC_CORES, m, D_FEAT), jnp.float32),
        mesh=mesh,
        scratch_types=[
            pltpu.VMEM((n_ch, chs), jnp.int32),
            pltpu.VMEM((KS, chs, D_FEAT), jnp.float32),
            pltpu.VMEM_SHARED((m, D_FEAT), jnp.float32),
            pltpu.SemaphoreType.DMA,
            pltpu.SemaphoreType.DMA,
        ],
        compiler_params=pltpu.CompilerParams(use_tc_tiling_on_sc=False),
    )
    def scatter_kernel(k_hbm, seg_hbm, z_hbm, out_hbm, seg_v, kr, acc,
                       sem_ld, sem_add):
        c = lax.axis_index("c")
        s = lax.axis_index("s")
        pltpu.sync_copy(z_hbm, acc.at[pl.ds(s * rows_per_tile, rows_per_tile)])
        plsc.subcore_barrier()
        base = c * per_core + s * per_w

        @pl.loop(0, n_ch // KS)
        def _(t):
            j0 = t * KS
            for b in range(KS):
                off = base + (j0 + b) * chs
                pltpu.async_copy(seg_hbm.at[pl.ds(off, chs)],
                                 seg_v.at[j0 + b], sem_ld)
                pltpu.async_copy(k_hbm.at[pl.ds(off, chs)], kr.at[b], sem_ld)
            for b in range(KS):
                pltpu.make_async_copy(seg_hbm.at[pl.ds(base, chs)],
                                      seg_v.at[0], sem_ld).wait()
                pltpu.make_async_copy(k_hbm.at[pl.ds(base, chs)],
                                      kr.at[b], sem_ld).wait()
            for b in range(KS):
                pltpu.async_copy(kr.at[b], acc.at[seg_v.at[j0 + b]],
                                 sem_add, add=True)
            for b in range(KS):
                pltpu.make_async_copy(kr.at[b], acc.at[pl.ds(0, chs)],
                                      sem_add).wait()

        plsc.subcore_barrier()
        pltpu.sync_copy(
            acc.at[pl.ds(s * rows_per_tile, rows_per_tile)],
            out_hbm.at[c].at[pl.ds(s * rows_per_tile, rows_per_tile)])

    return scatter_kernel(k_arr, seg_ord, zeros)


def _tc_combine(partials_list):
    m = partials_list[0].shape[1]
    rows = 1000
    nparts = len(partials_list)

    def body(*refs):
        o_ref = refs[-1]
        acc = refs[0][0] + refs[0][1]
        for r in refs[1:-1]:
            acc = acc + r[0] + r[1]
        o_ref[...] = acc

    return pl.pallas_call(
        body,
        grid=(m // rows,),
        in_specs=[pl.BlockSpec((2, rows, D_FEAT), lambda i: (0, i, 0))
                  for _ in range(nparts)],
        out_specs=pl.BlockSpec((rows, D_FEAT), lambda i: (i, 0)),
        out_shape=jax.ShapeDtypeStruct((m, D_FEAT), jnp.float32),
        compiler_params=pltpu.CompilerParams(
            dimension_semantics=("parallel",)),
    )(*partials_list)


def kernel(y, f_y, neighbors_index, neighbors_row_splits,
           W1, b1, W2, b2, W3, b3):
    E = neighbors_index.shape[0]
    m = neighbors_row_splits.shape[0] - 1
    n = y.shape[0]
    H = W2.shape[0]
    nb = E // BLOCK_E

    ind = jnp.zeros((E,), jnp.int32).at[neighbors_row_splits[1:-1]].add(1)
    seg = jnp.cumsum(ind).astype(jnp.int32)

    idx = neighbors_index
    ci = jnp.concatenate(
        [idx.reshape(nb, BLOCK_E), seg.reshape(nb, BLOCK_E) + n],
        axis=1).reshape(2 * E)
    z5 = jnp.zeros((n, 5), jnp.float32)
    z8 = jnp.zeros((n, 8), jnp.float32)
    big_table = jnp.concatenate(
        [jnp.concatenate([y, z5, z8], axis=1),
         jnp.concatenate([z8, y, z5], axis=1)], axis=0)

    W1all = jnp.concatenate(
        [W1[0:3],
         jnp.zeros((21, H), jnp.float32),
         W1[3:6],
         jnp.zeros((5, H), jnp.float32),
         W1[6:134],
         jnp.zeros((96, H), jnp.float32)],
        axis=0).astype(jnp.bfloat16)
    W2b = W2.astype(jnp.bfloat16)
    W3b = W3.astype(jnp.bfloat16)
    b1r = b1.reshape(1, H)
    b2r = b2.reshape(1, H)
    b3r = b3.reshape(1, D_FEAT)

    ec = E // NCHUNK
    partials = []
    for c in range(NCHUNK):
        Gc, Cc = _sc_gather(f_y, big_table,
                            lax.dynamic_slice_in_dim(idx, c * ec, ec),
                            lax.dynamic_slice_in_dim(ci, c * 2 * ec, 2 * ec))
        kc = _tc_mlp(Gc, Cc, W1all, W2b, W3b, b1r, b2r, b3r)
        partials.append(
            _sc_scatter(kc, lax.dynamic_slice_in_dim(seg, c * ec, ec), m))
    return _tc_combine(partials)


# speedup vs baseline: 1.0217x; 1.0217x over previous
import functools

import jax
import jax.numpy as jnp
from jax import lax
from jax.experimental import pallas as pl
from jax.experimental.pallas import tpu as pltpu
from jax.experimental.pallas import tpu_sc as plsc

N_SC_CORES = 2
N_SUBCORES = 16
NW = N_SC_CORES * N_SUBCORES
KG = 5
KC = 10
KS = 5
NCHUNK = 5

D_FEAT = 128
D_CPACK = 32
EDGES_PER_CROW = 128 // D_CPACK
CH = 80
BLOCK_E = 3200


def _sc_gather(f_y, big_table, idx, ci):
    E = idx.shape[0]
    per_w = E // NW
    n_g = per_w // CH
    ci_per_w = 2 * per_w
    n_c = ci_per_w // CH
    slots_per_block = 2 * BLOCK_E
    rows_per_block = BLOCK_E // EDGES_PER_CROW
    mesh = plsc.VectorSubcoreMesh(core_axis_name="c", subcore_axis_name="s")

    @functools.partial(
        pl.kernel,
        out_type=[
            jax.ShapeDtypeStruct((E, D_FEAT), jnp.float32),
            jax.ShapeDtypeStruct((E // EDGES_PER_CROW, 128), jnp.float32),
        ],
        mesh=mesh,
        scratch_types=[
            pltpu.VMEM((per_w,), jnp.int32),
            pltpu.VMEM((ci_per_w,), jnp.int32),
            pltpu.VMEM((KG, CH, D_FEAT), jnp.float32),
            pltpu.VMEM((KC, CH, 16), jnp.float32),
            pltpu.SemaphoreType.DMA,
            pltpu.SemaphoreType.DMA,
        ],
        compiler_params=pltpu.CompilerParams(use_tc_tiling_on_sc=False),
    )
    def gather_kernel(fy_hbm, bt_hbm, idx_hbm, ci_hbm, g_hbm, c_hbm,
                      idx_v, ci_v, gr, cr, sem_ld, sem_st):
        wid = lax.axis_index("s") * N_SC_CORES + lax.axis_index("c")
        base = wid * per_w
        cbase = wid * ci_per_w
        pltpu.sync_copy(idx_hbm.at[pl.ds(base, per_w)], idx_v)
        pltpu.sync_copy(ci_hbm.at[pl.ds(cbase, ci_per_w)], ci_v)

        @pl.loop(0, n_g // KG)
        def _(t):
            off0 = t * (KG * CH)
            for b in range(KG):
                pltpu.async_copy(
                    fy_hbm.at[idx_v.at[pl.ds(off0 + b * CH, CH)]],
                    gr.at[b], sem_ld)
            for b in range(KG):
                pltpu.make_async_copy(
                    fy_hbm.at[idx_v.at[pl.ds(0, CH)]], gr.at[b],
                    sem_ld).wait()
            for b in range(KG):
                pltpu.async_copy(
                    gr.at[b], g_hbm.at[pl.ds(base + off0 + b * CH, CH)],
                    sem_st)
            for b in range(KG):
                pltpu.make_async_copy(
                    gr.at[b], g_hbm.at[pl.ds(base, CH)], sem_st).wait()

        @pl.loop(0, n_c // KC)
        def _(t):
            off0 = t * (KC * CH)
            for b in range(KC):
                pltpu.async_copy(
                    bt_hbm.at[ci_v.at[pl.ds(off0 + b * CH, CH)]],
                    cr.at[b], sem_ld)
            for b in range(KC):
                pltpu.make_async_copy(
                    bt_hbm.at[ci_v.at[pl.ds(0, CH)]], cr.at[b],
                    sem_ld).wait()
            for b in range(KC):
                g0 = cbase + off0 + b * CH
                blk = g0 // slots_per_block
                rem = g0 % slots_per_block
                u = rem // rows_per_block
                r0 = rem % rows_per_block
                pltpu.async_copy(
                    cr.at[b],
                    c_hbm.at[pl.ds(blk * rows_per_block + r0, CH),
                             pl.ds(u * 16, 16)], sem_st)
            for b in range(KC):
                pltpu.make_async_copy(
                    cr.at[b], c_hbm.at[pl.ds(0, CH), pl.ds(0, 16)],
                    sem_st).wait()

    return gather_kernel(f_y, big_table, idx, ci)


def _gelu_bf16(x):
    xb = x.astype(jnp.bfloat16)
    c = jnp.bfloat16(0.7978845608028654)
    a = jnp.bfloat16(0.044715)
    t = jnp.tanh(c * (xb + a * (xb * xb * xb)))
    return jnp.bfloat16(0.5) * xb * (jnp.bfloat16(1.0) + t)


def _tc_mlp(G, C, W1all, W2, W3, b1, b2, b3):
    E = G.shape[0]
    H = W2.shape[0]
    bc = BLOCK_E // EDGES_PER_CROW

    def body(g_ref, c_ref, w1_ref, w2_ref, w3_ref,
             b1_ref, b2_ref, b3_ref, o_ref):
        g = g_ref[...]
        gb = g.astype(jnp.bfloat16)
        cpb = c_ref[...].astype(jnp.bfloat16)
        rep = jnp.concatenate(
            [cpb[:, 16 * u:16 * (u + 1)] for u in range(4)], axis=0)
        slf = jnp.concatenate(
            [cpb[:, 16 * u:16 * (u + 1)] for u in range(4, 8)], axis=0)
        x1 = jnp.concatenate(
            [rep, slf, gb,
             jnp.zeros((BLOCK_E, 96), jnp.bfloat16)], axis=1)
        h = jnp.dot(x1, w1_ref[...],
                    preferred_element_type=jnp.float32) + b1_ref[...]
        h = _gelu_bf16(h)
        h = jnp.dot(h, w2_ref[...],
                    preferred_element_type=jnp.float32) + b2_ref[...]
        h = _gelu_bf16(h)
        k = jnp.dot(h, w3_ref[...], preferred_element_type=jnp.float32)
        o_ref[...] = (k + b3_ref[...]) * g

    return pl.pallas_call(
        body,
        grid=(E // BLOCK_E,),
        in_specs=[
            pl.BlockSpec((BLOCK_E, D_FEAT), lambda i: (i, 0)),
            pl.BlockSpec((bc, 128), lambda i: (i, 0)),
            pl.BlockSpec((256, H), lambda i: (0, 0)),
            pl.BlockSpec((H, H), lambda i: (0, 0)),
            pl.BlockSpec((H, D_FEAT), lambda i: (0, 0)),
            pl.BlockSpec((1, H), lambda i: (0, 0)),
            pl.BlockSpec((1, H), lambda i: (0, 0)),
            pl.BlockSpec((1, D_FEAT), lambda i: (0, 0)),
        ],
        out_specs=pl.BlockSpec((BLOCK_E, D_FEAT), lambda i: (i, 0)),
        out_shape=jax.ShapeDtypeStruct((E, D_FEAT), jnp.float32),
        compiler_params=pltpu.CompilerParams(
            dimension_semantics=("parallel",)),
    )(G, C, W1all, W2, W3, b1, b2, b3)


def _sc_scatter(k_arr, seg_ord, m):
    E = k_arr.shape[0]
    per_core = E // N_SC_CORES
    per_w = per_core // N_SUBCORES
    chs = 40
    n_ch = per_w // chs
    rows_per_tile = m // N_SUBCORES
    mesh = plsc.VectorSubcoreMesh(core_axis_name="c", subcore_axis_name="s")
    zeros = jnp.zeros((rows_per_tile, D_FEAT), jnp.float32)

    @functools.partial(
        pl.kernel,
        out_type=jax.ShapeDtypeStruct((N_SC_CORES, m, D_FEAT), jnp.float32),
        mesh=mesh,
        scratch_types=[
            pltpu.VMEM((n_ch, chs), jnp.int32),
            pltpu.VMEM((KS, chs, D_FEAT), jnp.float32),
            pltpu.VMEM_SHARED((m, D_FEAT), jnp.float32),
            pltpu.SemaphoreType.DMA,
            pltpu.SemaphoreType.DMA,
        ],
        compiler_params=pltpu.CompilerParams(use_tc_tiling_on_sc=False),
    )
    def scatter_kernel(k_hbm, seg_hbm, z_hbm, out_hbm, seg_v, kr, acc,
                       sem_ld, sem_add):
        c = lax.axis_index("c")
        s = lax.axis_index("s")
        pltpu.sync_copy(z_hbm, acc.at[pl.ds(s * rows_per_tile, rows_per_tile)])
        plsc.subcore_barrier()
        base = c * per_core + s * per_w

        @pl.loop(0, n_ch // KS)
        def _(t):
            j0 = t * KS
            for b in range(KS):
                off = base + (j0 + b) * chs
                pltpu.async_copy(seg_hbm.at[pl.ds(off, chs)],
                                 seg_v.at[j0 + b], sem_ld)
                pltpu.async_copy(k_hbm.at[pl.ds(off, chs)], kr.at[b], sem_ld)
            for b in range(KS):
                pltpu.make_async_copy(seg_hbm.at[pl.ds(base, chs)],
                                      seg_v.at[0], sem_ld).wait()
                pltpu.make_async_copy(k_hbm.at[pl.ds(base, chs)],
                                      kr.at[b], sem_ld).wait()
            for b in range(KS):
                pltpu.async_copy(kr.at[b], acc.at[seg_v.at[j0 + b]],
                                 sem_add, add=True)
            for b in range(KS):
                pltpu.make_async_copy(kr.at[b], acc.at[pl.ds(0, chs)],
                                      sem_add).wait()

        plsc.subcore_barrier()
        pltpu.sync_copy(
            acc.at[pl.ds(s * rows_per_tile, rows_per_tile)],
            out_hbm.at[c].at[pl.ds(s * rows_per_tile, rows_per_tile)])

    return scatter_kernel(k_arr, seg_ord, zeros)


def _tc_combine(partials_list):
    m = partials_list[0].shape[1]
    rows = 1000
    nparts = len(partials_list)

    def body(*refs):
        o_ref = refs[-1]
        acc = refs[0][0] + refs[0][1]
        for r in refs[1:-1]:
            acc = acc + r[0] + r[1]
        o_ref[...] = acc

    return pl.pallas_call(
        body,
        grid=(m // rows,),
        in_specs=[pl.BlockSpec((2, rows, D_FEAT), lambda i: (0, i, 0))
                  for _ in range(nparts)],
        out_specs=pl.BlockSpec((rows, D_FEAT), lambda i: (i, 0)),
        out_shape=jax.ShapeDtypeStruct((m, D_FEAT), jnp.float32),
        compiler_params=pltpu.CompilerParams(
            dimension_semantics=("parallel",)),
    )(*partials_list)


def kernel(y, f_y, neighbors_index, neighbors_row_splits,
           W1, b1, W2, b2, W3, b3):
    E = neighbors_index.shape[0]
    m = neighbors_row_splits.shape[0] - 1
    n = y.shape[0]
    H = W2.shape[0]
    nb = E // BLOCK_E

    ind = jnp.zeros((E,), jnp.int32).at[neighbors_row_splits[1:-1]].add(1)
    seg = jnp.cumsum(ind).astype(jnp.int32)

    idx = neighbors_index
    ci = jnp.concatenate(
        [idx.reshape(nb, BLOCK_E), seg.reshape(nb, BLOCK_E) + n],
        axis=1).reshape(2 * E)
    z5 = jnp.zeros((n, 5), jnp.float32)
    z8 = jnp.zeros((n, 8), jnp.float32)
    big_table = jnp.concatenate(
        [jnp.concatenate([y, z5, z8], axis=1),
         jnp.concatenate([z8, y, z5], axis=1)], axis=0)

    W1all = jnp.concatenate(
        [W1[0:3],
         jnp.zeros((21, H), jnp.float32),
         W1[3:6],
         jnp.zeros((5, H), jnp.float32),
         W1[6:134],
         jnp.zeros((96, H), jnp.float32)],
        axis=0).astype(jnp.bfloat16)
    W2b = W2.astype(jnp.bfloat16)
    W3b = W3.astype(jnp.bfloat16)
    b1r = b1.reshape(1, H)
    b2r = b2.reshape(1, H)
    b3r = b3.reshape(1, D_FEAT)

    ec = E // NCHUNK
    partials = []
    for c in range(NCHUNK):
        Gc, Cc = _sc_gather(f_y, big_table,
                            lax.dynamic_slice_in_dim(idx, c * ec, ec),
                            lax.dynamic_slice_in_dim(ci, c * 2 * ec, 2 * ec))
        kc = _tc_mlp(Gc, Cc, W1all, W2b, W3b, b1r, b2r, b3r)
        partials.append(
            _sc_scatter(kc, lax.dynamic_slice_in_dim(seg, c * ec, ec), m))
    return _tc_combine(partials)
